# baseline (speedup 1.0000x reference)
import functools

import jax
import jax.numpy as jnp
from jax import lax
from jax.experimental import pallas as pl
from jax.experimental.pallas import tpu as pltpu
from jax.experimental.pallas import tpu_sc as plsc

NC, NS, L = 2, 16, 16
NW = NC * NS
B = 16384
D = 64
BPW = B // NW
CHUNK = 128
NCHUNK = BPW // CHUNK


@functools.partial(
    pl.kernel,
    out_type=jax.ShapeDtypeStruct((B, D), jnp.float32),
    mesh=plsc.VectorSubcoreMesh(core_axis_name="c", subcore_axis_name="s"),
    compiler_params=pltpu.CompilerParams(
        needs_layout_passes=False, use_tc_tiling_on_sc=False
    ),
    scratch_types=[
        pltpu.VMEM((BPW,), jnp.int32),
        pltpu.VMEM((BPW, D), jnp.float32),
        pltpu.SemaphoreType.DMA,
    ],
)
def _gather_softmax(table_hbm, idx_hbm, out_hbm, idx_v, rows_v, sem):
    wid = lax.axis_index("s") * NC + lax.axis_index("c")
    base = wid * BPW

    pltpu.sync_copy(idx_hbm.at[pl.ds(base, BPW)], idx_v)

    copies = []
    for c in range(NCHUNK):
        copies.append(
            pltpu.async_copy(
                table_hbm.at[idx_v.at[pl.ds(c * CHUNK, CHUNK)]],
                rows_v.at[pl.ds(c * CHUNK, CHUNK)],
                sem,
            )
        )
    for cp in copies:
        cp.wait()

    iota = lax.iota(jnp.int32, L)

    def body(g, carry):
        row_ids = g * L + iota
        cols = [
            plsc.load_gather(rows_v, [row_ids, jnp.full((L,), j, jnp.int32)])
            for j in range(D)
        ]
        m = cols[0]
        for j in range(1, D):
            m = jnp.maximum(m, cols[j])
        e = [jnp.exp(c - m) for c in cols]
        s = e[0]
        for j in range(1, D):
            s = s + e[j]
        inv = 1.0 / s
        for j in range(D):
            plsc.store_scatter(
                rows_v, [row_ids, jnp.full((L,), j, jnp.int32)], e[j] * inv
            )
        return carry

    lax.fori_loop(0, BPW // L, body, 0)

    pltpu.sync_copy(rows_v, out_hbm.at[pl.ds(base, BPW)])


def kernel(x, table):
    return _gather_softmax(table, x.astype(jnp.int32))


# speedup vs baseline: 1.5915x; 1.5915x over previous
import functools

import jax
import jax.numpy as jnp
from jax import lax
from jax.experimental import pallas as pl
from jax.experimental.pallas import tpu as pltpu
from jax.experimental.pallas import tpu_sc as plsc

NC, NS, L = 2, 16, 16
NW = NC * NS
B = 16384
D = 64
V = 1000000
BPW = B // NW
NGRP = BPW // L


@functools.partial(
    pl.kernel,
    out_type=jax.ShapeDtypeStruct((B, D), jnp.float32),
    mesh=plsc.VectorSubcoreMesh(core_axis_name="c", subcore_axis_name="s"),
    compiler_params=pltpu.CompilerParams(
        needs_layout_passes=False, use_tc_tiling_on_sc=True
    ),
    scratch_types=[
        pltpu.VMEM((BPW,), jnp.int32),
        pltpu.VMEM((BPW, D), jnp.float32),
        pltpu.SemaphoreType.DMA,
    ],
)
def _gather_softmax(tab_hbm, idx_hbm, out_hbm, idx_v, rows_v, sem):
    wid = lax.axis_index("s") * NC + lax.axis_index("c")
    base = wid * BPW

    pltpu.sync_copy(idx_hbm.at[pl.ds(base, BPW)], idx_v)

    def fetch_body(g, carry):
        o = pl.multiple_of(g * L, L)
        rvec = idx_v[pl.ds(o, L)]
        copies = []
        for k in range(L):
            copies.append(
                pltpu.async_copy(
                    tab_hbm.at[pl.ds(rvec[k], 1)],
                    rows_v.at[pl.ds(o + k, 1)],
                    sem,
                )
            )
        for cp in copies:
            cp.wait()
        return carry

    lax.fori_loop(0, NGRP, fetch_body, 0)

    iota16 = lax.iota(jnp.int32, L)

    def softmax_body(g, carry):
        slot = g * L + iota16
        cols = [
            plsc.load_gather(rows_v, [slot, jnp.full((L,), j, jnp.int32)])
            for j in range(D)
        ]
        m = cols[0]
        for j in range(1, D):
            m = jnp.maximum(m, cols[j])
        e = [jnp.exp(col - m) for col in cols]
        s = e[0]
        for j in range(1, D):
            s = s + e[j]
        inv = 1.0 / s
        for j in range(D):
            plsc.store_scatter(
                rows_v, [slot, jnp.full((L,), j, jnp.int32)], e[j] * inv
            )
        return carry

    lax.fori_loop(0, NGRP, softmax_body, 0)

    pltpu.sync_copy(rows_v, out_hbm.at[pl.ds(base, BPW)])


def kernel(x, table):
    return _gather_softmax(table, x.astype(jnp.int32))


# speedup vs baseline: 3.3520x; 2.1062x over previous
import functools

import jax
import jax.numpy as jnp
from jax import lax
from jax.experimental import pallas as pl
from jax.experimental.pallas import tpu as pltpu
from jax.experimental.pallas import tpu_sc as plsc

NC, NS, L = 2, 16, 16
NW = NC * NS
B = 16384
D = 64
V = 1000000
W = 1024
NWIN = 977
WPT = 31
NVB = B // L
CAP = 4096


@functools.partial(
    pl.kernel,
    out_type=jax.ShapeDtypeStruct((B, D), jnp.float32),
    mesh=plsc.VectorSubcoreMesh(core_axis_name="c", subcore_axis_name="s"),
    compiler_params=pltpu.CompilerParams(
        needs_layout_passes=False, use_tc_tiling_on_sc=True
    ),
    scratch_types=[
        pltpu.VMEM((B,), jnp.int32),
        pltpu.VMEM((B,), jnp.int32),
        pltpu.VMEM((B,), jnp.int32),
        pltpu.VMEM((CAP,), jnp.int32),
        pltpu.VMEM((CAP,), jnp.int32),
        pltpu.VMEM((D, W), jnp.float32),
        pltpu.VMEM((L, D), jnp.float32),
        pltpu.SemaphoreType.DMA,
        pltpu.SemaphoreType.DMA,
    ],
)
def _gather_softmax(tabT_hbm, tail_hbm, idx_hbm, out_hbm, idx_v, my_x, my_b,
                    wk_c, wk_b, slab, stage, sem, osem):
    t = lax.axis_index("s") * NC + lax.axis_index("c")
    iota = lax.iota(jnp.int32, L)

    pltpu.sync_copy(idx_hbm, idx_v)

    def zero_body(i, carry):
        wk_c[pl.ds(pl.multiple_of(i * L, L), L)] = jnp.zeros((L,), jnp.int32)
        return carry

    lax.fori_loop(0, CAP // L, zero_body, 0)

    def sel_body(i, off):
        o = pl.multiple_of(i * L, L)
        x = idx_v[pl.ds(o, L)]
        mine = (lax.shift_right_logical(x, 10) & 31) == t
        pos = off + plsc.cumsum(mine.astype(jnp.int32)) - 1
        plsc.store_scatter(my_x, [pos], x, mask=mine)
        plsc.store_scatter(my_b, [pos], o + iota, mask=mine)
        return off + plsc.all_reduce_population_count(mine)[0]

    n_my = lax.fori_loop(0, NVB, sel_body, jnp.int32(0))
    nv = lax.shift_right_logical(n_my + (L - 1), 4)

    def window_body(wi, carry):
        w = wi * NW + t

        @pl.when(w < NWIN - 1)
        def _full():
            base = w * W
            cps = [
                pltpu.async_copy(
                    tabT_hbm.at[pl.ds(8 * a, 8), pl.ds(base, W)],
                    slab.at[pl.ds(8 * a, 8), :],
                    sem,
                )
                for a in range(D // 8)
            ]
            for cp in cps:
                cp.wait()

        @pl.when(w == NWIN - 1)
        def _last():
            base = (NWIN - 1) * W
            cps = [
                pltpu.async_copy(
                    tabT_hbm.at[pl.ds(8 * a, 8), pl.ds(base, 512)],
                    slab.at[pl.ds(8 * a, 8), pl.ds(0, 512)],
                    sem,
                )
                for a in range(D // 8)
            ]
            cps.append(
                pltpu.async_copy(tail_hbm, slab.at[:, pl.ds(512, 128)], sem)
            )
            for cp in cps:
                cp.wait()

        @pl.when(w < NWIN)
        def _process():
            def wsel_body(i, woff):
                o = i * L
                x = my_x[pl.ds(o, L)]
                b = my_b[pl.ds(o, L)]
                sel = (lax.shift_right_logical(x, 10) == w) & (
                    (o + iota) < n_my
                )
                pos = woff + plsc.cumsum(sel.astype(jnp.int32)) - 1
                plsc.store_scatter(wk_c, [pos], x - w * W, mask=sel)
                plsc.store_scatter(wk_b, [pos], b, mask=sel)
                return woff + plsc.all_reduce_population_count(sel)[0]

            wcnt = lax.fori_loop(0, nv, wsel_body, jnp.int32(0))
            ng = lax.shift_right_logical(wcnt + (L - 1), 4)

            def group_body(g, carry2):
                o = g * L
                c16 = wk_c[pl.ds(o, L)]
                f = [
                    plsc.load_gather(slab, [jnp.full((L,), j, jnp.int32), c16])
                    for j in range(D)
                ]
                m = f[0]
                for j in range(1, D):
                    m = jnp.maximum(m, f[j])
                e = [jnp.exp(v - m) for v in f]
                s = e[0]
                for j in range(1, D):
                    s = s + e[j]
                inv = 1.0 / s
                for j in range(D):
                    plsc.store_scatter(
                        stage, [iota, jnp.full((L,), j, jnp.int32)],
                        e[j] * inv,
                    )
                b16 = wk_b[pl.ds(o, L)]
                for k in range(L):
                    @pl.when(o + k < wcnt)
                    def _emit():
                        pltpu.async_copy(
                            stage.at[pl.ds(k, 1)],
                            out_hbm.at[pl.ds(b16[k], 1)],
                            osem,
                        )

                def odrain_body(i, carry3):
                    pltpu.make_async_copy(
                        stage.at[pl.ds(0, 1)], out_hbm.at[pl.ds(0, 1)], osem
                    ).wait()
                    return carry3

                lax.fori_loop(0, jnp.minimum(wcnt - o, L), odrain_body, 0)
                return carry2

            lax.fori_loop(0, ng, group_body, 0)

        return carry

    lax.fori_loop(0, WPT, window_body, 0)


def kernel(x, table):
    tail = jnp.zeros((D, 128), jnp.float32).at[:, :64].set(table[999936:].T)
    return _gather_softmax(table.T, tail, x.astype(jnp.int32))


# speedup vs baseline: 4.1938x; 1.2511x over previous
import functools

import jax
import jax.numpy as jnp
from jax import lax
from jax.experimental import pallas as pl
from jax.experimental.pallas import tpu as pltpu
from jax.experimental.pallas import tpu_sc as plsc

NC, NS, L = 2, 16, 16
NW = NC * NS
B = 16384
D = 64
V = 1000000
W = 512
LASTW = V // W
NVB = B // L
PAIRS = 31
CAP = 4096


@functools.partial(
    pl.kernel,
    out_type=jax.ShapeDtypeStruct((B, D), jnp.float32),
    mesh=plsc.VectorSubcoreMesh(core_axis_name="c", subcore_axis_name="s"),
    compiler_params=pltpu.CompilerParams(
        needs_layout_passes=False, use_tc_tiling_on_sc=True
    ),
    scratch_types=[
        pltpu.VMEM((B,), jnp.int32),
        pltpu.VMEM((B,), jnp.int32),
        pltpu.VMEM((B,), jnp.int32),
        pltpu.VMEM((CAP,), jnp.int32),
        pltpu.VMEM((CAP,), jnp.int32),
        pltpu.VMEM((D, W), jnp.float32),
        pltpu.VMEM((D, W), jnp.float32),
        pltpu.VMEM((L, D), jnp.float32),
        pltpu.SemaphoreType.DMA,
        pltpu.SemaphoreType.DMA,
        pltpu.SemaphoreType.DMA,
    ],
)
def _gather_softmax(tabT_hbm, tail_hbm, idx_hbm, out_hbm, idx_v, my_x, my_b,
                    wk_c, wk_b, slab_a, slab_b, stage, sem_a, sem_b, osem):
    t = lax.axis_index("s") * NC + lax.axis_index("c")
    iota = lax.iota(jnp.int32, L)

    pltpu.sync_copy(idx_hbm, idx_v)

    def zero_body(i, carry):
        wk_c[pl.ds(pl.multiple_of(i * L, L), L)] = jnp.zeros((L,), jnp.int32)
        return carry

    lax.fori_loop(0, CAP // L, zero_body, 0)

    def sel_body(i, off):
        o = pl.multiple_of(i * L, L)
        x = idx_v[pl.ds(o, L)]
        mine = (lax.shift_right_logical(x, 9) & (NW - 1)) == t
        pos = off + plsc.cumsum(mine.astype(jnp.int32)) - 1
        plsc.store_scatter(my_x, [pos], x, mask=mine)
        plsc.store_scatter(my_b, [pos], o + iota, mask=mine)
        return off + plsc.all_reduce_population_count(mine)[0]

    n_my = lax.fori_loop(0, NVB, sel_body, jnp.int32(0))
    nv = lax.shift_right_logical(n_my + (L - 1), 4)

    def start_fetch(w, slab, sem):
        @pl.when(w < LASTW)
        def _full():
            base = w * W
            for a in range(D // 8):
                pltpu.async_copy(
                    tabT_hbm.at[pl.ds(8 * a, 8), pl.ds(base, W)],
                    slab.at[pl.ds(8 * a, 8), :],
                    sem,
                )

        @pl.when(w == LASTW)
        def _tail():
            pltpu.async_copy(tail_hbm, slab.at[:, pl.ds(0, 128)], sem)

    def wait_fetch(w, slab, sem):
        @pl.when(w < LASTW)
        def _full():
            for a in range(D // 8):
                pltpu.make_async_copy(
                    tabT_hbm.at[pl.ds(0, 8), pl.ds(0, W)],
                    slab.at[pl.ds(0, 8), :],
                    sem,
                ).wait()

        @pl.when(w == LASTW)
        def _tail():
            pltpu.make_async_copy(
                tail_hbm, slab.at[:, pl.ds(0, 128)], sem
            ).wait()

    def process(w, slab):
        @pl.when(w <= LASTW)
        def _():
            def wsel_body(i, woff):
                o = i * L
                x = my_x[pl.ds(o, L)]
                b = my_b[pl.ds(o, L)]
                sel = (lax.shift_right_logical(x, 9) == w) & (
                    (o + iota) < n_my
                )
                pos = woff + plsc.cumsum(sel.astype(jnp.int32)) - 1
                plsc.store_scatter(wk_c, [pos], x - w * W, mask=sel)
                plsc.store_scatter(wk_b, [pos], b, mask=sel)
                return woff + plsc.all_reduce_population_count(sel)[0]

            wcnt = lax.fori_loop(0, nv, wsel_body, jnp.int32(0))
            ng = lax.shift_right_logical(wcnt + (L - 1), 4)

            def group_body(g, carry2):
                o = g * L
                c16 = wk_c[pl.ds(o, L)]
                f = [
                    plsc.load_gather(slab, [jnp.full((L,), j, jnp.int32), c16])
                    for j in range(D)
                ]
                m = f[0]
                for j in range(1, D):
                    m = jnp.maximum(m, f[j])
                e = [jnp.exp(v - m) for v in f]
                s = e[0]
                for j in range(1, D):
                    s = s + e[j]
                inv = 1.0 / s
                for j in range(D):
                    plsc.store_scatter(
                        stage, [iota, jnp.full((L,), j, jnp.int32)],
                        e[j] * inv,
                    )
                b16 = wk_b[pl.ds(o, L)]
                for k in range(L):
                    @pl.when(o + k < wcnt)
                    def _emit():
                        pltpu.async_copy(
                            stage.at[pl.ds(k, 1)],
                            out_hbm.at[pl.ds(b16[k], 1)],
                            osem,
                        )

                def odrain_body(i, carry3):
                    pltpu.make_async_copy(
                        stage.at[pl.ds(0, 1)], out_hbm.at[pl.ds(0, 1)], osem
                    ).wait()
                    return carry3

                lax.fori_loop(0, jnp.minimum(wcnt - o, L), odrain_body, 0)
                return carry2

            lax.fori_loop(0, ng, group_body, 0)

    start_fetch(t, slab_a, sem_a)
    start_fetch(NW + t, slab_b, sem_b)

    def pair_body(i, carry):
        wa = (2 * i) * NW + t
        wb = (2 * i + 1) * NW + t
        wait_fetch(wa, slab_a, sem_a)
        process(wa, slab_a)
        start_fetch((2 * i + 2) * NW + t, slab_a, sem_a)
        wait_fetch(wb, slab_b, sem_b)
        process(wb, slab_b)
        start_fetch((2 * i + 3) * NW + t, slab_b, sem_b)
        return carry

    lax.fori_loop(0, PAIRS, pair_body, 0)


def kernel(x, table):
    tail = jnp.zeros((D, 128), jnp.float32).at[:, :64].set(table[999936:].T)
    return _gather_softmax(table.T, tail, x.astype(jnp.int32))
